# baseline (speedup 1.0000x reference)
import jax
import jax.numpy as jnp
from jax import lax
from jax.experimental import pallas as pl
from jax.experimental.pallas import tpu as pltpu
from jax.experimental.pallas import tpu_sc as plsc

_L = 16


def kernel(x, buffer, buffer_index):
    del x
    idx = buffer_index.astype(jnp.int32)
    mesh = plsc.VectorSubcoreMesh(
        core_axis_name="c", subcore_axis_name="s", num_cores=1, num_subcores=1
    )

    def body(buf_hbm, out_hbm, val_v, sem):
        idx_vec = lax.broadcast_in_dim(idx, (_L,), ())
        pltpu.async_copy(buf_hbm.at[idx_vec], val_v, sem).wait()
        pltpu.sync_copy(val_v, out_hbm)

    gather = pl.kernel(
        body,
        out_type=jax.ShapeDtypeStruct((_L,), jnp.float32),
        mesh=mesh,
        scratch_types=[
            pltpu.VMEM((_L,), jnp.float32),
            pltpu.SemaphoreType.DMA,
        ],
    )
    out = gather(buffer)
    return out[0]
